# Initial kernel scaffold; baseline (speedup 1.0000x reference)
import jax
import jax.numpy as jnp
from jax.experimental import pallas as pl


def kernel(item_idx, text_features, image_features):
    raise NotImplementedError("write your pallas kernel here")


# speedup vs baseline: 2.1957x; 2.1957x over previous
import functools

import jax
import jax.numpy as jnp
from jax import lax
from jax.experimental import pallas as pl
from jax.experimental.pallas import tpu as pltpu
from jax.experimental.pallas import tpu_sc as plsc

N_ITEMS = 100000
TEXT_DIM = 384
IMAGE_DIM = 512
OUT_DIM = TEXT_DIM + IMAGE_DIM
BATCH = 16384

_info = plsc.get_sparse_core_info()
_NC, _NS = _info.num_cores, _info.num_subcores
_NW = _NC * _NS
_B_PER_W = BATCH // _NW
_CHUNK = 128
_N_CHUNKS = _B_PER_W // _CHUNK


def _sc_gather_concat(idx_hbm, text_hbm, image_hbm, out_hbm,
                      idx_v, text_v, image_v, sem):
    wid = lax.axis_index("s") * _NC + lax.axis_index("c")
    base = wid * _B_PER_W
    pltpu.sync_copy(idx_hbm.at[pl.ds(base, _B_PER_W)], idx_v)
    for c in range(_N_CHUNKS):
        idx_chunk = idx_v.at[pl.ds(c * _CHUNK, _CHUNK)]
        ct = pltpu.async_copy(text_hbm.at[idx_chunk], text_v, sem)
        ci = pltpu.async_copy(image_hbm.at[idx_chunk], image_v, sem)
        ct.wait()
        ci.wait()
        row0 = base + c * _CHUNK
        pltpu.sync_copy(
            text_v, out_hbm.at[pl.ds(row0, _CHUNK), pl.ds(0, TEXT_DIM)])
        pltpu.sync_copy(
            image_v, out_hbm.at[pl.ds(row0, _CHUNK), pl.ds(TEXT_DIM, IMAGE_DIM)])


@jax.jit
def _encode(item_idx, text_features, image_features):
    mesh = plsc.VectorSubcoreMesh(core_axis_name="c", subcore_axis_name="s")
    run = functools.partial(
        pl.kernel,
        mesh=mesh,
        out_type=jax.ShapeDtypeStruct((BATCH, OUT_DIM), jnp.float32),
        scratch_types=[
            pltpu.VMEM((_B_PER_W,), jnp.int32),
            pltpu.VMEM((_CHUNK, TEXT_DIM), jnp.float32),
            pltpu.VMEM((_CHUNK, IMAGE_DIM), jnp.float32),
            pltpu.SemaphoreType.DMA,
        ],
    )(_sc_gather_concat)
    return run(item_idx.astype(jnp.int32), text_features, image_features)


def kernel(item_idx, text_features, image_features):
    return _encode(item_idx, text_features, image_features)


# speedup vs baseline: 2.2747x; 1.0360x over previous
import functools

import jax
import jax.numpy as jnp
from jax import lax
from jax.experimental import pallas as pl
from jax.experimental.pallas import tpu as pltpu
from jax.experimental.pallas import tpu_sc as plsc

N_ITEMS = 100000
TEXT_DIM = 384
IMAGE_DIM = 512
OUT_DIM = TEXT_DIM + IMAGE_DIM
BATCH = 16384

_info = plsc.get_sparse_core_info()
_NC, _NS = _info.num_cores, _info.num_subcores
_NW = _NC * _NS
_B_PER_W = BATCH // _NW
_CHUNK = 64
_N_CHUNKS = _B_PER_W // _CHUNK


def _sc_gather_concat(idx_hbm, text_hbm, image_hbm, out_hbm,
                      idx_v, comb0, comb1, gsem, wsem0, wsem1):
    wid = lax.axis_index("s") * _NC + lax.axis_index("c")
    base = wid * _B_PER_W
    pltpu.sync_copy(idx_hbm.at[pl.ds(base, _B_PER_W)], idx_v)
    combs = (comb0, comb1)
    wsems = (wsem0, wsem1)

    def fire_gather(c, buf):
        idx_chunk = idx_v.at[pl.ds(c * _CHUNK, _CHUNK)]
        ht = pltpu.async_copy(
            text_hbm.at[idx_chunk], buf.at[:, pl.ds(0, TEXT_DIM)], gsem)
        hi = pltpu.async_copy(
            image_hbm.at[idx_chunk], buf.at[:, pl.ds(TEXT_DIM, IMAGE_DIM)],
            gsem)
        return ht, hi

    handles = [fire_gather(0, comb0), fire_gather(1, comb1)]
    for c in range(_N_CHUNKS):
        b = c & 1
        ht, hi = handles[b]
        ht.wait()
        hi.wait()
        row0 = base + c * _CHUNK
        wh = pltpu.async_copy(
            combs[b], out_hbm.at[pl.ds(row0, _CHUNK)], wsems[b])
        if c + 2 < _N_CHUNKS:
            wh.wait()
            handles[b] = fire_gather(c + 2, combs[b])
        else:
            wh.wait()


@jax.jit
def _encode(item_idx, text_features, image_features):
    mesh = plsc.VectorSubcoreMesh(core_axis_name="c", subcore_axis_name="s")
    run = functools.partial(
        pl.kernel,
        mesh=mesh,
        out_type=jax.ShapeDtypeStruct((BATCH, OUT_DIM), jnp.float32),
        scratch_types=[
            pltpu.VMEM((_B_PER_W,), jnp.int32),
            pltpu.VMEM((_CHUNK, OUT_DIM), jnp.float32),
            pltpu.VMEM((_CHUNK, OUT_DIM), jnp.float32),
            pltpu.SemaphoreType.DMA,
            pltpu.SemaphoreType.DMA,
            pltpu.SemaphoreType.DMA,
        ],
    )(_sc_gather_concat)
    return run(item_idx.astype(jnp.int32), text_features, image_features)


def kernel(item_idx, text_features, image_features):
    return _encode(item_idx, text_features, image_features)
